# baseline (speedup 1.0000x reference)
import functools

import jax
import jax.numpy as jnp
from jax import lax
from jax.experimental import pallas as pl
from jax.experimental.pallas import tpu as pltpu

_PROPOSAL_COUNT = 1000
_PRE_NMS_LIMIT = 6000
_NMS_THRESHOLD = 0.7
_NEG_F = -1e9

_R = 160
_C = 128
_PAD_N = _R * _C


def _nms_body(scores_ref, deltas_ref, anch_ref, out_ref,
              sw_ref, by1_ref, bx1_ref, by2_ref, bx2_ref, areas_ref):
    B = scores_ref.shape[0]

    _NEG = jnp.float32(_NEG_F)
    ay1 = anch_ref[0]
    ax1 = anch_ref[1]
    ay2 = anch_ref[2]
    ax2 = anch_ref[3]
    dy = deltas_ref[0] * jnp.float32(0.1)
    dx = deltas_ref[1] * jnp.float32(0.1)
    dh = deltas_ref[2] * jnp.float32(0.2)
    dw = deltas_ref[3] * jnp.float32(0.2)
    h = ay2 - ay1
    w = ax2 - ax1
    cy = ay1 + jnp.float32(0.5) * h
    cx = ax1 + jnp.float32(0.5) * w
    cy = cy + dy * h
    cx = cx + dx * w
    h = h * jnp.exp(dh)
    w = w * jnp.exp(dw)
    y1 = cy - jnp.float32(0.5) * h
    x1 = cx - jnp.float32(0.5) * w
    y2 = y1 + h
    x2 = x1 + w
    one = jnp.float32(1.0)
    zero = jnp.float32(0.0)
    y1 = jnp.maximum(jnp.minimum(y1, one), zero)
    x1 = jnp.maximum(jnp.minimum(x1, one), zero)
    y2 = jnp.maximum(jnp.minimum(y2, one), zero)
    x2 = jnp.maximum(jnp.minimum(x2, one), zero)
    by1_ref[...] = y1
    bx1_ref[...] = x1
    by2_ref[...] = y2
    bx2_ref[...] = x2
    areas_ref[...] = (y2 - y1) * (x2 - x1)

    idx2d = (lax.broadcasted_iota(jnp.int32, (_R, _C), 0) * _C
             + lax.broadcasted_iota(jnp.int32, (_R, _C), 1))
    K = jnp.int32(_PRE_NMS_LIMIT)
    for b in range(B):
        sc = scores_ref[b]
        keys = lax.bitcast_convert_type(sc, jnp.int32)

        def bs_bits(_, carry):
            lo, hi = carry
            mid = (lo + hi) >> 1
            cnt = jnp.sum(jnp.where(keys >= mid, jnp.int32(1), jnp.int32(0)))
            ge = cnt >= K
            return jnp.where(ge, mid, lo), jnp.where(ge, hi, mid)

        v_lo, _ = lax.fori_loop(
            0, 31, bs_bits, (jnp.int32(0), jnp.int32(0x3F800000)))
        c_gt = jnp.sum(jnp.where(keys > v_lo, jnp.int32(1), jnp.int32(0)))
        need = K - c_gt
        eq = keys == v_lo

        def bs_idx(_, carry):
            lo_i, hi_i = carry
            mid = (lo_i + hi_i) >> 1
            cnt = jnp.sum(jnp.where(eq & (idx2d < mid),
                                    jnp.int32(1), jnp.int32(0)))
            ge = cnt >= need
            return jnp.where(ge, lo_i, mid), jnp.where(ge, mid, hi_i)

        _, i_hi = lax.fori_loop(
            0, 15, bs_idx, (jnp.int32(0), jnp.int32(_PAD_N)))
        mask = (keys > v_lo) | (eq & (idx2d < i_hi))
        sw_ref[b] = jnp.where(mask, sc, _NEG)

    lane = lax.broadcasted_iota(jnp.int32, (1, _C), 1)
    big = jnp.int32(1 << 30)
    thresh = jnp.float32(_NMS_THRESHOLD)
    eps = jnp.float32(1e-8)
    keep_floor = _NEG * jnp.float32(0.5)

    def step(i, _):
        for b in range(B):
            sw = sw_ref[b]
            m = jnp.max(sw, keepdims=True)
            mi = jnp.min(jnp.where(sw == m, idx2d, big), keepdims=True)
            oh = idx2d == mi
            y1p = by1_ref[b]
            x1p = bx1_ref[b]
            y2p = by2_ref[b]
            x2p = bx2_ref[b]
            b_y1 = jnp.sum(jnp.where(oh, y1p, zero), keepdims=True)
            b_x1 = jnp.sum(jnp.where(oh, x1p, zero), keepdims=True)
            b_y2 = jnp.sum(jnp.where(oh, y2p, zero), keepdims=True)
            b_x2 = jnp.sum(jnp.where(oh, x2p, zero), keepdims=True)
            area_best = (b_y2 - b_y1) * (b_x2 - b_x1)
            yy1 = jnp.maximum(b_y1, y1p)
            xx1 = jnp.maximum(b_x1, x1p)
            yy2 = jnp.minimum(b_y2, y2p)
            xx2 = jnp.minimum(b_x2, x2p)
            inter = (jnp.maximum(yy2 - yy1, zero)
                     * jnp.maximum(xx2 - xx1, zero))
            iou = inter / (areas_ref[b] + area_best - inter + eps)
            sw_ref[b] = jnp.where(iou >= thresh, _NEG, sw)

            keep = m > keep_floor
            vals = jnp.where(lane == 0, b_y1,
                             jnp.where(lane == 1, b_x1,
                                       jnp.where(lane == 2, b_y2, b_x2)))
            vals = jnp.where(keep, vals, zero)
            out_ref[b, pl.ds(i, 1), :] = vals
        return 0

    lax.fori_loop(0, _PROPOSAL_COUNT, step, 0)


@jax.jit
def kernel(rpn_class, rpn_bbox, anchors):
    B, N = rpn_class.shape[0], rpn_class.shape[1]
    pad = _PAD_N - N
    scores = rpn_class[:, :, 1]
    scores_p = jnp.pad(scores, ((0, 0), (0, pad)),
                       constant_values=-1.0).reshape(B, _R, _C)
    deltas_p = jnp.pad(jnp.transpose(rpn_bbox, (2, 0, 1)),
                       ((0, 0), (0, 0), (0, pad))).reshape(4, B, _R, _C)
    anch_p = jnp.pad(jnp.transpose(anchors, (2, 0, 1)),
                     ((0, 0), (0, 0), (0, pad))).reshape(4, B, _R, _C)

    plane = pltpu.VMEM((B, _R, _C), jnp.float32)
    out = pl.pallas_call(
        _nms_body,
        out_shape=jax.ShapeDtypeStruct((B, _PROPOSAL_COUNT, _C), jnp.float32),
        scratch_shapes=[plane] * 6,
    )(scores_p, deltas_p, anch_p)
    return out[:, :, :4]


# speedup vs baseline: 1.0665x; 1.0665x over previous
import functools

import jax
import jax.numpy as jnp
from jax import lax
from jax.experimental import pallas as pl
from jax.experimental.pallas import tpu as pltpu
from jax.experimental.pallas import tpu_sc as plsc

_PROPOSAL_COUNT = 1000
_K = 6000
_NMS_THRESHOLD = 0.7
_NEG_F = -1e9

_R = 160
_C = 128
_PAD_N = _R * _C
_B = 2

_GR = 48
_G = _GR * _C
_NF = 16
_NSUB = 32
_CHUNK = (_B * _PAD_N) // _NSUB
_NGROUP = _CHUNK // 128
_OUT_ROWS = _B * _G + _B * _R


def _select_body(scores_ref, sw_ref, dest_ref):
    NEG = jnp.float32(_NEG_F)
    idx2d = (lax.broadcasted_iota(jnp.int32, (_R, _C), 0) * _C
             + lax.broadcasted_iota(jnp.int32, (_R, _C), 1))
    row2d = lax.broadcasted_iota(jnp.int32, (_R, _C), 0)
    tri_c = (lax.broadcasted_iota(jnp.int32, (_C, _C), 0)
             <= lax.broadcasted_iota(jnp.int32, (_C, _C), 1)).astype(jnp.float32)
    tri_r = (lax.broadcasted_iota(jnp.int32, (_R, _R), 0)
             > lax.broadcasted_iota(jnp.int32, (_R, _R), 1)).astype(jnp.float32)
    K = jnp.int32(_K)
    for b in range(_B):
        sc = scores_ref[b]
        keys = lax.bitcast_convert_type(sc, jnp.int32)

        def bs_bits(_, carry):
            lo, hi = carry
            mid = (lo + hi) >> 1
            cnt = jnp.sum(jnp.where(keys >= mid, jnp.int32(1), jnp.int32(0)))
            ge = cnt >= K
            return jnp.where(ge, mid, lo), jnp.where(ge, hi, mid)

        v_lo, _ = lax.fori_loop(
            0, 31, bs_bits, (jnp.int32(0), jnp.int32(0x3F800000)))
        c_gt = jnp.sum(jnp.where(keys > v_lo, jnp.int32(1), jnp.int32(0)))
        need = K - c_gt
        eq = keys == v_lo

        def bs_idx(_, carry):
            lo_i, hi_i = carry
            mid = (lo_i + hi_i) >> 1
            cnt = jnp.sum(jnp.where(eq & (idx2d < mid),
                                    jnp.int32(1), jnp.int32(0)))
            ge = cnt >= need
            return jnp.where(ge, lo_i, mid), jnp.where(ge, mid, hi_i)

        _, i_hi = lax.fori_loop(
            0, 15, bs_idx, (jnp.int32(0), jnp.int32(_PAD_N)))
        mask = (keys > v_lo) | (eq & (idx2d < i_hi))
        sw_ref[b] = jnp.where(mask, sc, NEG)

        mf = mask.astype(jnp.float32)
        p_in = jnp.dot(mf, tri_c, preferred_element_type=jnp.float32)
        row_sums = p_in[:, _C - 1:_C]
        row_off = jnp.dot(tri_r, row_sums,
                          preferred_element_type=jnp.float32)
        rank = (p_in - mf) + row_off
        rank_i = rank.astype(jnp.int32)
        dest_ref[b] = jnp.where(
            mask, jnp.int32(b * _G) + rank_i,
            jnp.int32(_B * _G + b * _R) + row2d)


def _sc_scatter(rows, dest):
    info = plsc.get_sparse_core_info()
    mesh = plsc.VectorSubcoreMesh(core_axis_name="c", subcore_axis_name="s")

    @functools.partial(
        pl.kernel, mesh=mesh,
        compiler_params=pltpu.CompilerParams(use_tc_tiling_on_sc=False),
        out_type=jax.ShapeDtypeStruct((_OUT_ROWS, _NF), jnp.float32),
        scratch_types=[
            pltpu.VMEM((_CHUNK, _NF), jnp.float32),
            pltpu.VMEM((_NGROUP, 128), jnp.int32),
            pltpu.SemaphoreType.DMA,
        ],
    )
    def body(rows_hbm, dest_hbm, out_hbm, rows_v, idx_v, sem):
        w = lax.axis_index("s") * info.num_cores + lax.axis_index("c")
        base = w * _CHUNK
        pltpu.sync_copy(rows_hbm.at[pl.ds(base, _CHUNK)], rows_v)
        pltpu.sync_copy(dest_hbm.at[w], idx_v)
        cps = [
            pltpu.async_copy(rows_v.at[pl.ds(j * 128, 128)],
                             out_hbm.at[idx_v.at[j]], sem)
            for j in range(_NGROUP)
        ]
        for cp in cps:
            cp.wait()

    return body(rows, dest)


def _nms_body(g_ref, out_ref, sw_ref, by1_ref, bx1_ref, by2_ref, bx2_ref,
              areas_ref):
    NEG = jnp.float32(_NEG_F)
    zero = jnp.float32(0.0)
    one = jnp.float32(1.0)
    idx2d = (lax.broadcasted_iota(jnp.int32, (_GR, _C), 0) * _C
             + lax.broadcasted_iota(jnp.int32, (_GR, _C), 1))
    for b in range(_B):
        dy = g_ref[1, b] * jnp.float32(0.1)
        dx = g_ref[2, b] * jnp.float32(0.1)
        dh = g_ref[3, b] * jnp.float32(0.2)
        dw = g_ref[4, b] * jnp.float32(0.2)
        ay1 = g_ref[5, b]
        ax1 = g_ref[6, b]
        ay2 = g_ref[7, b]
        ax2 = g_ref[8, b]
        h = ay2 - ay1
        w = ax2 - ax1
        cy = ay1 + jnp.float32(0.5) * h
        cx = ax1 + jnp.float32(0.5) * w
        cy = cy + dy * h
        cx = cx + dx * w
        h = h * jnp.exp(dh)
        w = w * jnp.exp(dw)
        y1 = cy - jnp.float32(0.5) * h
        x1 = cx - jnp.float32(0.5) * w
        y2 = y1 + h
        x2 = x1 + w
        y1 = jnp.maximum(jnp.minimum(y1, one), zero)
        x1 = jnp.maximum(jnp.minimum(x1, one), zero)
        y2 = jnp.maximum(jnp.minimum(y2, one), zero)
        x2 = jnp.maximum(jnp.minimum(x2, one), zero)
        by1_ref[b] = y1
        bx1_ref[b] = x1
        by2_ref[b] = y2
        bx2_ref[b] = x2
        areas_ref[b] = (y2 - y1) * (x2 - x1)
        sw_ref[b] = jnp.where(idx2d < _K, g_ref[0, b], NEG)

    lane = lax.broadcasted_iota(jnp.int32, (1, _C), 1)
    big = jnp.int32(1 << 30)
    thresh = jnp.float32(_NMS_THRESHOLD)
    eps = jnp.float32(1e-8)
    keep_floor = NEG * jnp.float32(0.5)

    def step(i, _):
        for b in range(_B):
            sw = sw_ref[b]
            m = jnp.max(sw)
            bi = jnp.min(jnp.where(sw == m, idx2d, big))
            r = lax.shift_right_logical(bi, 7)
            c = lax.bitwise_and(bi, jnp.int32(127))
            oh = lane == c

            def ext(ref):
                row = ref[b, pl.ds(r, 1), :]
                return jnp.sum(jnp.where(oh, row, zero))

            b_y1 = ext(by1_ref)
            b_x1 = ext(bx1_ref)
            b_y2 = ext(by2_ref)
            b_x2 = ext(bx2_ref)
            area_best = (b_y2 - b_y1) * (b_x2 - b_x1)
            yy1 = jnp.maximum(b_y1, by1_ref[b])
            xx1 = jnp.maximum(b_x1, bx1_ref[b])
            yy2 = jnp.minimum(b_y2, by2_ref[b])
            xx2 = jnp.minimum(b_x2, bx2_ref[b])
            inter = (jnp.maximum(yy2 - yy1, zero)
                     * jnp.maximum(xx2 - xx1, zero))
            iou = inter / (areas_ref[b] + area_best - inter + eps)
            sw_ref[b] = jnp.where(iou >= thresh, NEG, sw)

            keep = m > keep_floor
            vals = jnp.where(lane == 0, b_y1,
                             jnp.where(lane == 1, b_x1,
                                       jnp.where(lane == 2, b_y2, b_x2)))
            vals = jnp.where(keep, vals, zero)
            out_ref[b, pl.ds(i, 1), :] = vals
        return 0

    lax.fori_loop(0, _PROPOSAL_COUNT, step, 0)


@jax.jit
def kernel(rpn_class, rpn_bbox, anchors):
    B, N = rpn_class.shape[0], rpn_class.shape[1]
    pad = _PAD_N - N
    scores_p = jnp.pad(rpn_class[:, :, 1], ((0, 0), (0, pad)),
                       constant_values=-1.0).reshape(B, _R, _C)

    sw, dest = pl.pallas_call(
        _select_body,
        out_shape=(jax.ShapeDtypeStruct((B, _R, _C), jnp.float32),
                   jax.ShapeDtypeStruct((B, _R, _C), jnp.int32)),
    )(scores_p)

    fields = jnp.concatenate(
        [sw.reshape(B, _PAD_N, 1),
         jnp.pad(rpn_bbox, ((0, 0), (0, pad), (0, 0))),
         jnp.pad(anchors, ((0, 0), (0, pad), (0, 0)))], axis=2)
    rows = jnp.pad(fields, ((0, 0), (0, 0), (0, _NF - 9))).reshape(
        B * _PAD_N, _NF)
    dest_c = dest.reshape(_NSUB, _NGROUP, 128)

    packed = _sc_scatter(rows, dest_c)

    g = packed[:_B * _G].reshape(_B, _G, _NF)
    g = jnp.transpose(g, (2, 0, 1)).reshape(_NF, _B, _GR, _C)

    plane = pltpu.VMEM((_B, _GR, _C), jnp.float32)
    out = pl.pallas_call(
        _nms_body,
        out_shape=jax.ShapeDtypeStruct((_B, _PROPOSAL_COUNT, _C), jnp.float32),
        scratch_shapes=[plane] * 6,
    )(g)
    return out[:, :, :4]


# speedup vs baseline: 1.9768x; 1.8536x over previous
import functools

import jax
import jax.numpy as jnp
from jax import lax
from jax.experimental import pallas as pl
from jax.experimental.pallas import tpu as pltpu
from jax.experimental.pallas import tpu_sc as plsc

_PROPOSAL_COUNT = 1000
_K = 6000
_NMS_THRESHOLD = 0.7
_NEG_F = -1e9

_R = 160
_C = 128
_PAD_N = _R * _C
_B = 2

_GR = 48
_G = _GR * _C
_NF = 16
_NSUB = 32
_CHUNK = (_B * _PAD_N) // _NSUB
_NGROUP = _CHUNK // 128
_OUT_ROWS = _B * _G + _B * _R


def _select_body(scores_ref, sw_ref, dest_ref):
    NEG = jnp.float32(_NEG_F)
    idx2d = (lax.broadcasted_iota(jnp.int32, (_R, _C), 0) * _C
             + lax.broadcasted_iota(jnp.int32, (_R, _C), 1))
    row2d = lax.broadcasted_iota(jnp.int32, (_R, _C), 0)
    tri_c = (lax.broadcasted_iota(jnp.int32, (_C, _C), 0)
             <= lax.broadcasted_iota(jnp.int32, (_C, _C), 1)).astype(jnp.float32)
    tri_r = (lax.broadcasted_iota(jnp.int32, (_R, _R), 0)
             > lax.broadcasted_iota(jnp.int32, (_R, _R), 1)).astype(jnp.float32)
    K = jnp.int32(_K)
    for b in range(_B):
        sc = scores_ref[b]
        keys = lax.bitcast_convert_type(sc, jnp.int32)

        def bs_bits(_, carry):
            lo, hi = carry
            mid = (lo + hi) >> 1
            cnt = jnp.sum(jnp.where(keys >= mid, jnp.int32(1), jnp.int32(0)))
            ge = cnt >= K
            return jnp.where(ge, mid, lo), jnp.where(ge, hi, mid)

        v_lo, _ = lax.fori_loop(
            0, 31, bs_bits, (jnp.int32(0), jnp.int32(0x3F800000)))
        c_gt = jnp.sum(jnp.where(keys > v_lo, jnp.int32(1), jnp.int32(0)))
        need = K - c_gt
        eq = keys == v_lo

        def bs_idx(_, carry):
            lo_i, hi_i = carry
            mid = (lo_i + hi_i) >> 1
            cnt = jnp.sum(jnp.where(eq & (idx2d < mid),
                                    jnp.int32(1), jnp.int32(0)))
            ge = cnt >= need
            return jnp.where(ge, lo_i, mid), jnp.where(ge, mid, hi_i)

        _, i_hi = lax.fori_loop(
            0, 15, bs_idx, (jnp.int32(0), jnp.int32(_PAD_N)))
        mask = (keys > v_lo) | (eq & (idx2d < i_hi))
        sw_ref[b] = jnp.where(mask, sc, NEG)

        mf = mask.astype(jnp.float32)
        p_in = jnp.dot(mf, tri_c, preferred_element_type=jnp.float32)
        row_sums = p_in[:, _C - 1:_C]
        row_off = jnp.dot(tri_r, row_sums,
                          preferred_element_type=jnp.float32)
        rank = (p_in - mf) + row_off
        rank_i = rank.astype(jnp.int32)
        dest_ref[b] = jnp.where(
            mask, jnp.int32(b * _G) + rank_i,
            jnp.int32(_B * _G + b * _R) + row2d)


def _sc_scatter(rows, dest):
    info = plsc.get_sparse_core_info()
    mesh = plsc.VectorSubcoreMesh(core_axis_name="c", subcore_axis_name="s")

    @functools.partial(
        pl.kernel, mesh=mesh,
        compiler_params=pltpu.CompilerParams(use_tc_tiling_on_sc=False),
        out_type=jax.ShapeDtypeStruct((_OUT_ROWS, _NF), jnp.float32),
        scratch_types=[
            pltpu.VMEM((_CHUNK, _NF), jnp.float32),
            pltpu.VMEM((_NGROUP, 128), jnp.int32),
            pltpu.SemaphoreType.DMA,
        ],
    )
    def body(rows_hbm, dest_hbm, out_hbm, rows_v, idx_v, sem):
        w = lax.axis_index("s") * info.num_cores + lax.axis_index("c")
        base = w * _CHUNK
        pltpu.sync_copy(rows_hbm.at[pl.ds(base, _CHUNK)], rows_v)
        pltpu.sync_copy(dest_hbm.at[w], idx_v)
        cps = [
            pltpu.async_copy(rows_v.at[pl.ds(j * 128, 128)],
                             out_hbm.at[idx_v.at[j]], sem)
            for j in range(_NGROUP)
        ]
        for cp in cps:
            cp.wait()

    return body(rows, dest)


def _nms_body(g_ref, out_ref, sw_ref, by1_ref, bx1_ref, by2_ref, bx2_ref,
              areas_ref):
    NEG = jnp.float32(_NEG_F)
    zero = jnp.float32(0.0)
    one = jnp.float32(1.0)
    idx3d = jnp.broadcast_to(
        lax.broadcasted_iota(jnp.int32, (_GR, _C), 0) * _C
        + lax.broadcasted_iota(jnp.int32, (_GR, _C), 1), (_B, _GR, _C))

    dy = g_ref[1] * jnp.float32(0.1)
    dx = g_ref[2] * jnp.float32(0.1)
    dh = g_ref[3] * jnp.float32(0.2)
    dw = g_ref[4] * jnp.float32(0.2)
    ay1 = g_ref[5]
    ax1 = g_ref[6]
    ay2 = g_ref[7]
    ax2 = g_ref[8]
    h = ay2 - ay1
    w = ax2 - ax1
    cy = ay1 + jnp.float32(0.5) * h
    cx = ax1 + jnp.float32(0.5) * w
    cy = cy + dy * h
    cx = cx + dx * w
    h = h * jnp.exp(dh)
    w = w * jnp.exp(dw)
    y1 = cy - jnp.float32(0.5) * h
    x1 = cx - jnp.float32(0.5) * w
    y2 = y1 + h
    x2 = x1 + w
    y1 = jnp.maximum(jnp.minimum(y1, one), zero)
    x1 = jnp.maximum(jnp.minimum(x1, one), zero)
    y2 = jnp.maximum(jnp.minimum(y2, one), zero)
    x2 = jnp.maximum(jnp.minimum(x2, one), zero)
    by1_ref[...] = y1
    bx1_ref[...] = x1
    by2_ref[...] = y2
    bx2_ref[...] = x2
    areas_ref[...] = (y2 - y1) * (x2 - x1)
    sw_ref[...] = jnp.where(idx3d < _K, g_ref[0], NEG)

    lane3 = lax.broadcasted_iota(jnp.int32, (1, 1, _C), 2)
    big = jnp.int32(1 << 30)
    thresh = jnp.float32(_NMS_THRESHOLD)
    eps = jnp.float32(1e-8)
    keep_floor = NEG * jnp.float32(0.5)

    def _red(x, op):
        t = op(x, axis=2, keepdims=True)
        return op(t, axis=1, keepdims=True)

    def step(i, _):
        sw = sw_ref[...]
        m = _red(sw, jnp.max)
        mi = _red(jnp.where(sw == m, idx3d, big), jnp.min)
        oh = idx3d == mi
        y1p = by1_ref[...]
        x1p = bx1_ref[...]
        y2p = by2_ref[...]
        x2p = bx2_ref[...]
        b_y1 = _red(jnp.where(oh, y1p, zero), jnp.sum)
        b_x1 = _red(jnp.where(oh, x1p, zero), jnp.sum)
        b_y2 = _red(jnp.where(oh, y2p, zero), jnp.sum)
        b_x2 = _red(jnp.where(oh, x2p, zero), jnp.sum)
        area_best = (b_y2 - b_y1) * (b_x2 - b_x1)
        yy1 = jnp.maximum(b_y1, y1p)
        xx1 = jnp.maximum(b_x1, x1p)
        yy2 = jnp.minimum(b_y2, y2p)
        xx2 = jnp.minimum(b_x2, x2p)
        inter = (jnp.maximum(yy2 - yy1, zero)
                 * jnp.maximum(xx2 - xx1, zero))
        iou = inter / (areas_ref[...] + area_best - inter + eps)
        sw_ref[...] = jnp.where(iou >= thresh, NEG, sw)

        keep = m > keep_floor
        vals = jnp.where(lane3 == 0, b_y1,
                         jnp.where(lane3 == 1, b_x1,
                                   jnp.where(lane3 == 2, b_y2, b_x2)))
        vals = jnp.where(keep, vals, zero)
        out_ref[:, pl.ds(i, 1), :] = vals
        return 0

    lax.fori_loop(0, _PROPOSAL_COUNT, step, 0)


@jax.jit
def kernel(rpn_class, rpn_bbox, anchors):
    B, N = rpn_class.shape[0], rpn_class.shape[1]
    pad = _PAD_N - N
    scores_p = jnp.pad(rpn_class[:, :, 1], ((0, 0), (0, pad)),
                       constant_values=-1.0).reshape(B, _R, _C)

    sw, dest = pl.pallas_call(
        _select_body,
        out_shape=(jax.ShapeDtypeStruct((B, _R, _C), jnp.float32),
                   jax.ShapeDtypeStruct((B, _R, _C), jnp.int32)),
    )(scores_p)

    fields = jnp.concatenate(
        [sw.reshape(B, _PAD_N, 1),
         jnp.pad(rpn_bbox, ((0, 0), (0, pad), (0, 0))),
         jnp.pad(anchors, ((0, 0), (0, pad), (0, 0)))], axis=2)
    rows = jnp.pad(fields, ((0, 0), (0, 0), (0, _NF - 9))).reshape(
        B * _PAD_N, _NF)
    dest_c = dest.reshape(_NSUB, _NGROUP, 128)

    packed = _sc_scatter(rows, dest_c)

    g = packed[:_B * _G].reshape(_B, _G, _NF)
    g = jnp.transpose(g, (2, 0, 1)).reshape(_NF, _B, _GR, _C)

    plane = pltpu.VMEM((_B, _GR, _C), jnp.float32)
    out = pl.pallas_call(
        _nms_body,
        out_shape=jax.ShapeDtypeStruct((_B, _PROPOSAL_COUNT, _C), jnp.float32),
        scratch_shapes=[plane] * 6,
    )(g)
    return out[:, :, :4]
